# Initial kernel scaffold; baseline (speedup 1.0000x reference)
import jax
import jax.numpy as jnp
from jax.experimental import pallas as pl


def kernel(ufeats, ifeats, enc_edge_index, enc_edge_type, dec_edge_index, W_u, W_i, W_out_u, W_out_i, Q, coef):
    raise NotImplementedError("write your pallas kernel here")


# speedup vs baseline: 14.1959x; 14.1959x over previous
import functools

import jax
import jax.numpy as jnp
from jax import lax
from jax.experimental import pallas as pl
from jax.experimental.pallas import tpu as pltpu
from jax.experimental.pallas import tpu_sc as plsc

R = 5
N = 5000
D = 128
HALF = 64
HID = 256
OUT = 64
RN = R * N
ACC = 25600
HROWS = ACC // 2
GROWS = 256
TSL = HROWS // 16
CH = 80
ZR = 80

_MESH = plsc.VectorSubcoreMesh(core_axis_name="c", subcore_axis_name="s")


def _sc_degrees(nodes_cat, typ):
    E = typ.shape[0]
    per_tile = E // 16
    n_chunks = per_tile // CH
    DSL = ACC // 16

    @functools.partial(
        pl.kernel,
        out_type=jax.ShapeDtypeStruct((2 * ACC,), jnp.float32),
        mesh=_MESH,
        scratch_types=[
            pltpu.VMEM((CH,), jnp.int32),
            pltpu.VMEM((CH,), jnp.int32),
            pltpu.VMEM((CH,), jnp.int32),
            pltpu.VMEM((CH,), jnp.float32),
            pltpu.VMEM((DSL,), jnp.float32),
            pltpu.VMEM_SHARED((ACC,), jnp.float32),
        ],
    )
    def deg_kernel(nodes_hbm, typ_hbm, out_hbm, nv, tv, iv, ones, zrow, acc):
        c = lax.axis_index("c")
        s = lax.axis_index("s")

        def zfill(i, _):
            zrow[pl.ds(i * 16, 16)] = jnp.zeros((16,), jnp.float32)
            return 0
        lax.fori_loop(0, DSL // 16, zfill, 0)
        for j in range(CH // 16):
            ones[pl.ds(j * 16, 16)] = jnp.ones((16,), jnp.float32)
        pltpu.sync_copy(zrow, acc.at[pl.ds(s * DSL, DSL)])
        plsc.subcore_barrier()

        base0 = s * per_tile

        def chunk(k, _):
            base = base0 + k * CH
            pltpu.sync_copy(nodes_hbm.at[pl.ds(c * E + base, CH)], nv)
            pltpu.sync_copy(typ_hbm.at[pl.ds(base, CH)], tv)
            for j in range(CH // 16):
                sl = pl.ds(j * 16, 16)
                iv[sl] = nv[sl] * R + tv[sl]
            pltpu.sync_copy(ones, acc.at[iv], add=True)
            return 0
        lax.fori_loop(0, n_chunks, chunk, 0)
        plsc.subcore_barrier()
        pltpu.sync_copy(acc.at[pl.ds(s * DSL, DSL)], zrow)
        pltpu.sync_copy(zrow, out_hbm.at[pl.ds(c * ACC + s * DSL, DSL)])

    return deg_kernel(nodes_cat, typ)


def _sc_edge_pass(gnodes, snodes, typ, table):
    E = typ.shape[0]
    per_tile = E // 16
    n_chunks = per_tile // CH

    @functools.partial(
        pl.kernel,
        out_type=jax.ShapeDtypeStruct((2, HROWS, D), jnp.float32),
        mesh=_MESH,
        scratch_types=[
            pltpu.VMEM((CH,), jnp.int32),
            pltpu.VMEM((CH,), jnp.int32),
            pltpu.VMEM((CH,), jnp.int32),
            pltpu.VMEM((CH,), jnp.int32),
            pltpu.VMEM((CH,), jnp.int32),
            pltpu.VMEM((CH, D), jnp.float32),
            pltpu.VMEM((ZR, D), jnp.float32),
            pltpu.VMEM_SHARED((HROWS + GROWS, D), jnp.float32),
            pltpu.SemaphoreType.DMA,
        ],
    )
    def edge_kernel(gn_hbm, sn_hbm, typ_hbm, tab_hbm, out_hbm,
                    gv, sv, tv, gi, si, rows, zblk, acc, sem):
        c = lax.axis_index("c")
        s = lax.axis_index("s")

        def zfill(i, _):
            for j in range(D // 16):
                zblk[i, pl.ds(j * 16, 16)] = jnp.zeros((16,), jnp.float32)
            return 0
        lax.fori_loop(0, ZR, zfill, 0)
        for k in range(TSL // ZR):
            pltpu.sync_copy(zblk, acc.at[pl.ds(s * TSL + k * ZR, ZR)])
        plsc.subcore_barrier()

        base0 = s * per_tile
        row0 = c * HROWS

        def chunk(k, _):
            base = base0 + k * CH
            pltpu.sync_copy(gn_hbm.at[pl.ds(base, CH)], gv)
            pltpu.sync_copy(sn_hbm.at[pl.ds(base, CH)], sv)
            pltpu.sync_copy(typ_hbm.at[pl.ds(base, CH)], tv)
            for j in range(CH // 16):
                sl = pl.ds(j * 16, 16)
                t = tv[sl]
                gi[sl] = gv[sl] * R + t
                loc = sv[sl] * R + t - row0
                oob = (loc < 0) | (loc >= HROWS)
                si[sl] = jnp.where(oob, HROWS + (sv[sl] & (GROWS - 1)), loc)
            pltpu.async_copy(tab_hbm.at[gi], rows, sem).wait()
            pltpu.sync_copy(rows, acc.at[si], add=True)
            return 0
        lax.fori_loop(0, n_chunks, chunk, 0)
        plsc.subcore_barrier()
        for k in range(TSL // ZR):
            off = s * TSL + k * ZR
            pltpu.sync_copy(acc.at[pl.ds(off, ZR)], zblk)
            pltpu.sync_copy(zblk, out_hbm.at[c, pl.ds(off, ZR)])

    return edge_kernel(gnodes, snodes, typ, table)


def _sc_dec_gather(z, dsrc, ddst):
    E = dsrc.shape[0]
    per_w = E // 32
    n_chunks = per_w // CH

    @functools.partial(
        pl.kernel,
        out_type=[jax.ShapeDtypeStruct((E, D), jnp.float32),
                  jax.ShapeDtypeStruct((E, D), jnp.float32)],
        mesh=_MESH,
        scratch_types=[
            pltpu.VMEM((CH,), jnp.int32),
            pltpu.VMEM((CH,), jnp.int32),
            pltpu.VMEM((CH, D), jnp.float32),
            pltpu.VMEM((CH, D), jnp.float32),
            pltpu.SemaphoreType.DMA,
            pltpu.SemaphoreType.DMA,
        ],
    )
    def dec_kernel(z_hbm, src_hbm, dst_hbm, w1_hbm, w2_hbm,
                   sv, dv, ru, rv, sem_u, sem_v):
        c = lax.axis_index("c")
        s = lax.axis_index("s")
        w = s * 2 + c
        base0 = w * per_w

        def chunk(k, _):
            base = base0 + k * CH
            pltpu.sync_copy(src_hbm.at[pl.ds(base, CH)], sv)
            pltpu.sync_copy(dst_hbm.at[pl.ds(base, CH)], dv)
            cp_u = pltpu.async_copy(z_hbm.at[sv], ru, sem_u)
            cp_v = pltpu.async_copy(z_hbm.at[dv], rv, sem_v)
            cp_u.wait()
            pltpu.sync_copy(ru, w1_hbm.at[pl.ds(base, CH)])
            cp_v.wait()
            pltpu.sync_copy(rv, w2_hbm.at[pl.ds(base, CH)])
            return 0
        lax.fori_loop(0, n_chunks, chunk, 0)

    return dec_kernel(z, dsrc, ddst)


def _tc_prep(deg_u, deg_i, ufeats, ifeats):
    NB = 200
    grid = (N // NB,)

    def body(du_ref, di_ref, uf_ref, if_ref,
             su_ref, si_ref, cu_ref, ci_ref):
        cu = lax.rsqrt(jnp.maximum(du_ref[...], 1.0))
        ci = lax.rsqrt(jnp.maximum(di_ref[...], 1.0))
        cu_ref[...] = cu
        ci_ref[...] = ci
        su_ref[...] = cu[:, :, None] * uf_ref[...][:, None, :]
        si_ref[...] = ci[:, :, None] * if_ref[...][:, None, :]

    deg_spec = pl.BlockSpec((NB, R), lambda i: (i, 0))
    feat_spec = pl.BlockSpec((NB, D), lambda i: (i, 0))
    tab_spec = pl.BlockSpec((NB, R, D), lambda i: (i, 0, 0))
    tab_ty = jax.ShapeDtypeStruct((N, R, D), jnp.float32)
    return pl.pallas_call(
        body,
        grid=grid,
        in_specs=[deg_spec, deg_spec, feat_spec, feat_spec],
        out_specs=[tab_spec, tab_spec, deg_spec, deg_spec],
        out_shape=[tab_ty, tab_ty,
                   jax.ShapeDtypeStruct((N, R), jnp.float32),
                   jax.ShapeDtypeStruct((N, R), jnp.float32)],
    )(deg_u, deg_i, ufeats, ifeats)


def _tc_encoder(acc_u, acc_i, cu, ci, W_u, W_i, W_out_u, W_out_i):
    NB = 200
    grid = (N // NB,)

    def body(au_ref, ai_ref, cu_ref, ci_ref, wu_ref, wi_ref,
             wou_ref, woi_ref, zu_ref, zi_ref):
        xu = cu_ref[...][:, :, None] * au_ref[...]
        xi = ci_ref[...][:, :, None] * ai_ref[...]
        wu = wu_ref[...]
        wi = wi_ref[...]
        agg_u = jnp.zeros((NB, HID), jnp.float32)
        agg_i = jnp.zeros((NB, HID), jnp.float32)
        for r in range(R):
            agg_u = agg_u + jax.lax.dot(xu[:, r, :], wi[r],
                                        preferred_element_type=jnp.float32)
            agg_i = agg_i + jax.lax.dot(xi[:, r, :], wu[r],
                                        preferred_element_type=jnp.float32)
        hu = jnp.maximum(agg_u, 0.0)
        hi = jnp.maximum(agg_i, 0.0)
        zu_ref[...] = jnp.maximum(
            jax.lax.dot(hu, wou_ref[...], preferred_element_type=jnp.float32), 0.0)
        zi_ref[...] = jnp.maximum(
            jax.lax.dot(hi, woi_ref[...], preferred_element_type=jnp.float32), 0.0)

    acc_spec = pl.BlockSpec((NB, R, D), lambda i: (i, 0, 0))
    c_spec = pl.BlockSpec((NB, R), lambda i: (i, 0))
    w_spec = pl.BlockSpec((R, D, HID), lambda i: (0, 0, 0))
    wo_spec = pl.BlockSpec((HID, OUT), lambda i: (0, 0))
    z_spec = pl.BlockSpec((NB, OUT), lambda i: (i, 0))
    z_ty = jax.ShapeDtypeStruct((N, OUT), jnp.float32)
    return pl.pallas_call(
        body,
        grid=grid,
        in_specs=[acc_spec, acc_spec, c_spec, c_spec, w_spec, w_spec,
                  wo_spec, wo_spec],
        out_specs=[z_spec, z_spec],
        out_shape=[z_ty, z_ty],
    )(acc_u, acc_i, cu, ci, W_u, W_i, W_out_u, W_out_i)


def _tc_decoder(w1, w2, Q, coef):
    E = w1.shape[0]
    EB = 1000
    grid = (E // EB,)

    def body(w1_ref, w2_ref, q_ref, coef_ref, out_ref):
        u = w1_ref[...][:, :OUT]
        v = w2_ref[...][:, OUT:]
        q0 = q_ref[0]
        q1 = q_ref[1]
        cols = []
        for r in range(R):
            p_r = coef_ref[r, 0] * q0 + coef_ref[r, 1] * q1
            t = jax.lax.dot_general(v, p_r, (((1,), (1,)), ((), ())),
                                    preferred_element_type=jnp.float32)
            cols.append(jnp.sum(u * t, axis=1))
        out_ref[...] = jnp.stack(cols, axis=1)

    w_spec = pl.BlockSpec((EB, D), lambda i: (i, 0))
    return pl.pallas_call(
        body,
        grid=grid,
        in_specs=[w_spec, w_spec,
                  pl.BlockSpec((2, OUT, OUT), lambda i: (0, 0, 0)),
                  pl.BlockSpec((R, 2), lambda i: (0, 0))],
        out_specs=pl.BlockSpec((EB, R), lambda i: (i, 0)),
        out_shape=jax.ShapeDtypeStruct((E, R), jnp.float32),
    )(w1, w2, Q, coef)


def kernel(ufeats, ifeats, enc_edge_index, enc_edge_type, dec_edge_index,
           W_u, W_i, W_out_u, W_out_i, Q, coef):
    src = enc_edge_index[0]
    dst = enc_edge_index[1]
    typ = enc_edge_type

    deg2 = _sc_degrees(jnp.concatenate([src, dst]), typ)
    deg_u = deg2[:RN].reshape(N, R)
    deg_i = deg2[ACC:ACC + RN].reshape(N, R)

    table_u, table_i, cu, ci = _tc_prep(deg_u, deg_i, ufeats, ifeats)
    table_u = table_u.reshape(RN, D)
    table_i = table_i.reshape(RN, D)

    acc_i2 = _sc_edge_pass(src, dst, typ, table_u)
    acc_u2 = _sc_edge_pass(dst, src, typ, table_i)
    acc_i = acc_i2.reshape(ACC, D)[:RN].reshape(N, R, D)
    acc_u = acc_u2.reshape(ACC, D)[:RN].reshape(N, R, D)

    zu, zi = _tc_encoder(acc_u, acc_i, cu, ci, W_u, W_i, W_out_u, W_out_i)
    z = jnp.concatenate([zu, zi], axis=1)
    w1, w2 = _sc_dec_gather(z, dec_edge_index[0], dec_edge_index[1])
    return _tc_decoder(w1, w2, Q, coef)


# speedup vs baseline: 26.7931x; 1.8874x over previous
import functools

import jax
import jax.numpy as jnp
from jax import lax
from jax.experimental import pallas as pl
from jax.experimental.pallas import tpu as pltpu
from jax.experimental.pallas import tpu_sc as plsc

R = 5
N = 5000
D = 128
HALF = 64
HID = 256
OUT = 64
RN = R * N
ACC = 25600
HROWS = ACC // 2
GROWS = 256
TSL = HROWS // 16
CH = 80
SEC = 128
ZR = 80

_MESH = plsc.VectorSubcoreMesh(core_axis_name="c", subcore_axis_name="s")


def _sc_degrees(pack, n_edges):
    E = n_edges
    per_tile = E // 16
    n_chunks = per_tile // CH
    DSL = ACC // 16
    PW = 3 * SEC

    @functools.partial(
        pl.kernel,
        out_type=jax.ShapeDtypeStruct((2 * ACC,), jnp.float32),
        mesh=_MESH,
        scratch_types=[
            pltpu.VMEM((2, PW), jnp.int32),
            pltpu.VMEM((2, CH), jnp.int32),
            pltpu.VMEM((CH,), jnp.float32),
            pltpu.VMEM((DSL,), jnp.float32),
            pltpu.VMEM_SHARED((ACC,), jnp.float32),
            pltpu.SemaphoreType.DMA,
        ],
    )
    def deg_kernel(pack_hbm, out_hbm, pvb, ivb, ones, zrow, acc, sem_i):
        c = lax.axis_index("c")
        s = lax.axis_index("s")

        def zfill(i, _):
            zrow[pl.ds(i * 16, 16)] = jnp.zeros((16,), jnp.float32)
            return 0
        lax.fori_loop(0, DSL // 16, zfill, 0)
        for j in range(CH // 16):
            ones[pl.ds(j * 16, 16)] = jnp.ones((16,), jnp.float32)
        pltpu.sync_copy(zrow, acc.at[pl.ds(s * DSL, DSL)])
        plsc.subcore_barrier()

        ck0 = s * n_chunks
        noff = c * SEC

        def fire_idx(k, b):
            pltpu.async_copy(pack_hbm.at[pl.ds((ck0 + k) * PW, PW)],
                             pvb.at[b], sem_i)

        def wait_idx(b):
            pltpu.make_async_copy(pack_hbm.at[pl.ds(0, PW)], pvb.at[b],
                                  sem_i).wait()

        def do_chunk(b):
            for j in range(CH // 16):
                sl = pl.ds(j * 16, 16)
                ivb[b, sl] = (pvb[b, pl.ds(noff + j * 16, 16)] * R
                              + pvb[b, pl.ds(2 * SEC + j * 16, 16)])
            pltpu.sync_copy(ones, acc.at[ivb.at[b]], add=True)

        M = n_chunks // 2
        fire_idx(0, 0)

        def pair(m, _):
            a = 2 * m
            wait_idx(0)
            fire_idx(a + 1, 1)
            do_chunk(0)
            wait_idx(1)

            @pl.when(m < M - 1)
            def _():
                fire_idx(a + 2, 0)
            do_chunk(1)
            return 0
        lax.fori_loop(0, M, pair, 0)
        plsc.subcore_barrier()
        pltpu.sync_copy(acc.at[pl.ds(s * DSL, DSL)], zrow)
        pltpu.sync_copy(zrow, out_hbm.at[pl.ds(c * ACC + s * DSL, DSL)])

    return deg_kernel(pack)


def _sc_edge_pass(pack, table, n_edges, swap):
    E = n_edges
    per_tile = E // 16
    n_chunks = per_tile // CH
    PW = 3 * SEC
    goff = SEC if swap else 0
    soff = 0 if swap else SEC

    @functools.partial(
        pl.kernel,
        out_type=jax.ShapeDtypeStruct((2, HROWS, D), jnp.float32),
        mesh=_MESH,
        scratch_types=[
            pltpu.VMEM((2, PW), jnp.int32),
            pltpu.VMEM((2, CH), jnp.int32),
            pltpu.VMEM((2, CH), jnp.int32),
            pltpu.VMEM((2, CH, D), jnp.float32),
            pltpu.VMEM_SHARED((HROWS + GROWS, D), jnp.float32),
            pltpu.SemaphoreType.DMA,
            pltpu.SemaphoreType.DMA,
            pltpu.SemaphoreType.DMA,
        ],
    )
    def edge_kernel(pack_hbm, tab_hbm, out_hbm,
                    pvb, gib, sib, rows, acc, sem_i, sem_g0, sem_g1):
        c = lax.axis_index("c")
        s = lax.axis_index("s")
        r0 = rows.at[0]

        def zfill(i, _):
            for j in range(D // 16):
                r0[i, pl.ds(j * 16, 16)] = jnp.zeros((16,), jnp.float32)
            return 0
        lax.fori_loop(0, ZR, zfill, 0)
        for k in range(TSL // ZR):
            pltpu.sync_copy(r0, acc.at[pl.ds(s * TSL + k * ZR, ZR)])
        plsc.subcore_barrier()

        ck0 = s * n_chunks
        row0 = c * HROWS

        def fire_idx(k, b):
            pltpu.async_copy(pack_hbm.at[pl.ds((ck0 + k) * PW, PW)],
                             pvb.at[b], sem_i)

        def wait_idx(b):
            pltpu.make_async_copy(pack_hbm.at[pl.ds(0, PW)], pvb.at[b],
                                  sem_i).wait()

        def compute_idx(b):
            for j in range(CH // 16):
                sl = pl.ds(j * 16, 16)
                t = pvb[b, pl.ds(2 * SEC + j * 16, 16)]
                gn = pvb[b, pl.ds(goff + j * 16, 16)]
                sn = pvb[b, pl.ds(soff + j * 16, 16)]
                gib[b, sl] = gn * R + t
                loc = sn * R + t - row0
                oob = (loc < 0) | (loc >= HROWS)
                sib[b, sl] = jnp.where(oob, HROWS + (sn & (GROWS - 1)), loc)

        def fire_gather(b, sem):
            pltpu.async_copy(tab_hbm.at[gib.at[b]], rows.at[b], sem)

        def wait_gather(b, sem):
            pltpu.make_async_copy(tab_hbm.at[gib.at[b]], rows.at[b],
                                  sem).wait()

        def scatter(b):
            pltpu.sync_copy(rows.at[b], acc.at[sib.at[b]], add=True)

        M = n_chunks // 2
        fire_idx(0, 0)

        def pair(m, _):
            a = 2 * m
            wait_idx(0)
            compute_idx(0)
            fire_gather(0, sem_g0)
            fire_idx(a + 1, 1)

            @pl.when(m > 0)
            def _():
                wait_gather(1, sem_g1)
                scatter(1)
            wait_idx(1)
            compute_idx(1)
            fire_gather(1, sem_g1)

            @pl.when(m < M - 1)
            def _():
                fire_idx(a + 2, 0)
            wait_gather(0, sem_g0)
            scatter(0)
            return 0
        lax.fori_loop(0, M, pair, 0)
        wait_gather(1, sem_g1)
        scatter(1)
        plsc.subcore_barrier()
        for k in range(TSL // ZR):
            off = s * TSL + k * ZR
            pltpu.sync_copy(acc.at[pl.ds(off, ZR)], r0)
            pltpu.sync_copy(r0, out_hbm.at[c, pl.ds(off, ZR)])

    return edge_kernel(pack, table)


def _sc_dec_gather(z, pack, n_edges):
    E = n_edges
    per_w = E // 32
    n_chunks = per_w // CH
    PW = 2 * SEC

    @functools.partial(
        pl.kernel,
        out_type=[jax.ShapeDtypeStruct((E, D), jnp.float32),
                  jax.ShapeDtypeStruct((E, D), jnp.float32)],
        mesh=_MESH,
        scratch_types=[
            pltpu.VMEM((2, PW), jnp.int32),
            pltpu.VMEM((2, CH), jnp.int32),
            pltpu.VMEM((2, CH), jnp.int32),
            pltpu.VMEM((2, CH, D), jnp.float32),
            pltpu.VMEM((2, CH, D), jnp.float32),
            pltpu.SemaphoreType.DMA,
            pltpu.SemaphoreType.DMA,
            pltpu.SemaphoreType.DMA,
            pltpu.SemaphoreType.DMA,
            pltpu.SemaphoreType.DMA,
        ],
    )
    def dec_kernel(z_hbm, pack_hbm, w1_hbm, w2_hbm,
                   pvb, svb, dvb, ru, rv, sem_i, sem_u0, sem_u1, sem_v0,
                   sem_v1):
        c = lax.axis_index("c")
        s = lax.axis_index("s")
        w = s * 2 + c
        ck0 = w * n_chunks
        base0 = w * per_w

        def fire_idx(k, b):
            pltpu.async_copy(pack_hbm.at[pl.ds((ck0 + k) * PW, PW)],
                             pvb.at[b], sem_i)

        def wait_idx(b):
            pltpu.make_async_copy(pack_hbm.at[pl.ds(0, PW)], pvb.at[b],
                                  sem_i).wait()
            for j in range(CH // 16):
                sl = pl.ds(j * 16, 16)
                svb[b, sl] = pvb[b, pl.ds(j * 16, 16)]
                dvb[b, sl] = pvb[b, pl.ds(SEC + j * 16, 16)]

        def fire_gathers(b, su, sv_):
            pltpu.async_copy(z_hbm.at[svb.at[b]], ru.at[b], su)
            pltpu.async_copy(z_hbm.at[dvb.at[b]], rv.at[b], sv_)

        def drain_write(k, b, su, sv_):
            base = base0 + k * CH
            pltpu.make_async_copy(z_hbm.at[svb.at[b]], ru.at[b], su).wait()
            pltpu.sync_copy(ru.at[b], w1_hbm.at[pl.ds(base, CH)])
            pltpu.make_async_copy(z_hbm.at[dvb.at[b]], rv.at[b], sv_).wait()
            pltpu.sync_copy(rv.at[b], w2_hbm.at[pl.ds(base, CH)])

        M = n_chunks // 2
        fire_idx(0, 0)

        def pair(m, _):
            a = 2 * m
            wait_idx(0)
            fire_gathers(0, sem_u0, sem_v0)
            fire_idx(a + 1, 1)

            @pl.when(m > 0)
            def _():
                drain_write(a - 1, 1, sem_u1, sem_v1)
            wait_idx(1)
            fire_gathers(1, sem_u1, sem_v1)

            @pl.when(m < M - 1)
            def _():
                fire_idx(a + 2, 0)
            drain_write(a, 0, sem_u0, sem_v0)
            return 0
        lax.fori_loop(0, M, pair, 0)
        drain_write(2 * M - 1, 1, sem_u1, sem_v1)
        if n_chunks % 2:
            k = n_chunks - 1
            fire_idx(k, 0)
            wait_idx(0)
            fire_gathers(0, sem_u0, sem_v0)
            drain_write(k, 0, sem_u0, sem_v0)

    return dec_kernel(z, pack)


def _tc_prep(deg_u, deg_i, ufeats, ifeats):
    NB = 200
    grid = (N // NB,)

    def body(du_ref, di_ref, uf_ref, if_ref,
             su_ref, si_ref, cu_ref, ci_ref):
        cu = lax.rsqrt(jnp.maximum(du_ref[...], 1.0))
        ci = lax.rsqrt(jnp.maximum(di_ref[...], 1.0))
        cu_ref[...] = cu
        ci_ref[...] = ci
        su_ref[...] = cu[:, :, None] * uf_ref[...][:, None, :]
        si_ref[...] = ci[:, :, None] * if_ref[...][:, None, :]

    deg_spec = pl.BlockSpec((NB, R), lambda i: (i, 0))
    feat_spec = pl.BlockSpec((NB, D), lambda i: (i, 0))
    tab_spec = pl.BlockSpec((NB, R, D), lambda i: (i, 0, 0))
    tab_ty = jax.ShapeDtypeStruct((N, R, D), jnp.float32)
    return pl.pallas_call(
        body,
        grid=grid,
        in_specs=[deg_spec, deg_spec, feat_spec, feat_spec],
        out_specs=[tab_spec, tab_spec, deg_spec, deg_spec],
        out_shape=[tab_ty, tab_ty,
                   jax.ShapeDtypeStruct((N, R), jnp.float32),
                   jax.ShapeDtypeStruct((N, R), jnp.float32)],
    )(deg_u, deg_i, ufeats, ifeats)


def _tc_encoder(acc_u, acc_i, cu, ci, W_u, W_i, W_out_u, W_out_i):
    NB = 200
    grid = (N // NB,)

    def body(au_ref, ai_ref, cu_ref, ci_ref, wu_ref, wi_ref,
             wou_ref, woi_ref, zu_ref, zi_ref):
        xu = cu_ref[...][:, :, None] * au_ref[...]
        xi = ci_ref[...][:, :, None] * ai_ref[...]
        wu = wu_ref[...]
        wi = wi_ref[...]
        agg_u = jnp.zeros((NB, HID), jnp.float32)
        agg_i = jnp.zeros((NB, HID), jnp.float32)
        for r in range(R):
            agg_u = agg_u + jax.lax.dot(xu[:, r, :], wi[r],
                                        preferred_element_type=jnp.float32)
            agg_i = agg_i + jax.lax.dot(xi[:, r, :], wu[r],
                                        preferred_element_type=jnp.float32)
        hu = jnp.maximum(agg_u, 0.0)
        hi = jnp.maximum(agg_i, 0.0)
        zu_ref[...] = jnp.maximum(
            jax.lax.dot(hu, wou_ref[...], preferred_element_type=jnp.float32), 0.0)
        zi_ref[...] = jnp.maximum(
            jax.lax.dot(hi, woi_ref[...], preferred_element_type=jnp.float32), 0.0)

    acc_spec = pl.BlockSpec((NB, R, D), lambda i: (i, 0, 0))
    c_spec = pl.BlockSpec((NB, R), lambda i: (i, 0))
    w_spec = pl.BlockSpec((R, D, HID), lambda i: (0, 0, 0))
    wo_spec = pl.BlockSpec((HID, OUT), lambda i: (0, 0))
    z_spec = pl.BlockSpec((NB, OUT), lambda i: (i, 0))
    z_ty = jax.ShapeDtypeStruct((N, OUT), jnp.float32)
    return pl.pallas_call(
        body,
        grid=grid,
        in_specs=[acc_spec, acc_spec, c_spec, c_spec, w_spec, w_spec,
                  wo_spec, wo_spec],
        out_specs=[z_spec, z_spec],
        out_shape=[z_ty, z_ty],
    )(acc_u, acc_i, cu, ci, W_u, W_i, W_out_u, W_out_i)


def _tc_decoder(w1, w2, Q, coef):
    E = w1.shape[0]
    EB = 1000
    grid = (E // EB,)

    def body(w1_ref, w2_ref, q_ref, coef_ref, out_ref):
        u = w1_ref[...][:, :OUT]
        v = w2_ref[...][:, OUT:]
        ys = []
        for b in range(2):
            t = jax.lax.dot_general(v, q_ref[b], (((1,), (1,)), ((), ())),
                                    preferred_element_type=jnp.float32)
            ys.append(jnp.sum(u * t, axis=1))
        out_ref[...] = (ys[0][:, None] * coef_ref[:, 0][None, :]
                        + ys[1][:, None] * coef_ref[:, 1][None, :])

    w_spec = pl.BlockSpec((EB, D), lambda i: (i, 0))
    return pl.pallas_call(
        body,
        grid=grid,
        in_specs=[w_spec, w_spec,
                  pl.BlockSpec((2, OUT, OUT), lambda i: (0, 0, 0)),
                  pl.BlockSpec((R, 2), lambda i: (0, 0))],
        out_specs=pl.BlockSpec((EB, R), lambda i: (i, 0)),
        out_shape=jax.ShapeDtypeStruct((E, R), jnp.float32),
    )(w1, w2, Q, coef)


def kernel(ufeats, ifeats, enc_edge_index, enc_edge_type, dec_edge_index,
           W_u, W_i, W_out_u, W_out_i, Q, coef):
    src = enc_edge_index[0]
    dst = enc_edge_index[1]
    typ = enc_edge_type
    E = typ.shape[0]
    nck = E // CH

    pad = ((0, 0), (0, SEC - CH))
    enc_pack = jnp.concatenate(
        [jnp.pad(src.reshape(nck, CH), pad),
         jnp.pad(dst.reshape(nck, CH), pad),
         jnp.pad(typ.reshape(nck, CH), pad)], axis=1).reshape(-1)
    ED = dec_edge_index.shape[1]
    dnck = ED // CH
    dec_pack = jnp.concatenate(
        [jnp.pad(dec_edge_index[0].reshape(dnck, CH), pad),
         jnp.pad(dec_edge_index[1].reshape(dnck, CH), pad)],
        axis=1).reshape(-1)

    deg2 = _sc_degrees(enc_pack, E)
    deg_u = deg2[:RN].reshape(N, R)
    deg_i = deg2[ACC:ACC + RN].reshape(N, R)

    table_u, table_i, cu, ci = _tc_prep(deg_u, deg_i, ufeats, ifeats)
    table_u = table_u.reshape(RN, D)
    table_i = table_i.reshape(RN, D)

    acc_i2 = _sc_edge_pass(enc_pack, table_u, E, swap=False)
    acc_u2 = _sc_edge_pass(enc_pack, table_i, E, swap=True)
    acc_i = acc_i2.reshape(ACC, D)[:RN].reshape(N, R, D)
    acc_u = acc_u2.reshape(ACC, D)[:RN].reshape(N, R, D)

    zu, zi = _tc_encoder(acc_u, acc_i, cu, ci, W_u, W_i, W_out_u, W_out_i)
    z = jnp.concatenate([zu, zi], axis=1)
    w1, w2 = _sc_dec_gather(z, dec_pack, ED)
    return _tc_decoder(w1, w2, Q, coef)


# speedup vs baseline: 27.8781x; 1.0405x over previous
import functools

import jax
import jax.numpy as jnp
from jax import lax
from jax.experimental import pallas as pl
from jax.experimental.pallas import tpu as pltpu
from jax.experimental.pallas import tpu_sc as plsc

R = 5
N = 5000
D = 128
HALF = 64
HID = 256
OUT = 64
RN = R * N
ACC = 25600
HROWS = ACC // 2
GROWS = 256
TSL = HROWS // 16
CH = 80
SEC = 128
ZR = 80

_MESH = plsc.VectorSubcoreMesh(core_axis_name="c", subcore_axis_name="s")


def _sc_degrees(pack, n_edges):
    E = n_edges
    per_tile = E // 16
    n_chunks = per_tile // CH
    DSL = ACC // 16
    PW = 3 * SEC

    @functools.partial(
        pl.kernel,
        out_type=jax.ShapeDtypeStruct((2 * ACC,), jnp.float32),
        mesh=_MESH,
        scratch_types=[
            pltpu.VMEM((2, PW), jnp.int32),
            pltpu.VMEM((2, CH), jnp.int32),
            pltpu.VMEM((CH,), jnp.float32),
            pltpu.VMEM((DSL,), jnp.float32),
            pltpu.VMEM_SHARED((ACC,), jnp.float32),
            pltpu.SemaphoreType.DMA,
        ],
    )
    def deg_kernel(pack_hbm, out_hbm, pvb, ivb, ones, zrow, acc, sem_i):
        c = lax.axis_index("c")
        s = lax.axis_index("s")

        def zfill(i, _):
            zrow[pl.ds(i * 16, 16)] = jnp.zeros((16,), jnp.float32)
            return 0
        lax.fori_loop(0, DSL // 16, zfill, 0)
        for j in range(CH // 16):
            ones[pl.ds(j * 16, 16)] = jnp.ones((16,), jnp.float32)
        pltpu.sync_copy(zrow, acc.at[pl.ds(s * DSL, DSL)])
        plsc.subcore_barrier()

        ck0 = s * n_chunks
        noff = c * SEC

        def fire_idx(k, b):
            pltpu.async_copy(pack_hbm.at[pl.ds((ck0 + k) * PW, PW)],
                             pvb.at[b], sem_i)

        def wait_idx(b):
            pltpu.make_async_copy(pack_hbm.at[pl.ds(0, PW)], pvb.at[b],
                                  sem_i).wait()

        def do_chunk(b):
            for j in range(CH // 16):
                sl = pl.ds(j * 16, 16)
                ivb[b, sl] = (pvb[b, pl.ds(noff + j * 16, 16)] * R
                              + pvb[b, pl.ds(2 * SEC + j * 16, 16)])
            pltpu.sync_copy(ones, acc.at[ivb.at[b]], add=True)

        M = n_chunks // 2
        fire_idx(0, 0)

        def pair(m, _):
            a = 2 * m
            wait_idx(0)
            fire_idx(a + 1, 1)
            do_chunk(0)
            wait_idx(1)

            @pl.when(m < M - 1)
            def _():
                fire_idx(a + 2, 0)
            do_chunk(1)
            return 0
        lax.fori_loop(0, M, pair, 0)
        plsc.subcore_barrier()
        pltpu.sync_copy(acc.at[pl.ds(s * DSL, DSL)], zrow)
        pltpu.sync_copy(zrow, out_hbm.at[pl.ds(c * ACC + s * DSL, DSL)])

    return deg_kernel(pack)


def _sc_edge_dual(pack, table_u, table_i, n_edges):
    E = n_edges
    per_tile = E // 16
    n_chunks = per_tile // CH
    PW = 3 * SEC

    @functools.partial(
        pl.kernel,
        out_type=[jax.ShapeDtypeStruct((2, HROWS, D), jnp.float32),
                  jax.ShapeDtypeStruct((2, HROWS, D), jnp.float32)],
        mesh=_MESH,
        scratch_types=[
            pltpu.VMEM((2, PW), jnp.int32),
            pltpu.VMEM((2, CH), jnp.int32),
            pltpu.VMEM((2, CH), jnp.int32),
            pltpu.VMEM((2, CH, D), jnp.float32),
            pltpu.VMEM_SHARED((HROWS + GROWS, D), jnp.float32),
            pltpu.SemaphoreType.DMA,
            pltpu.SemaphoreType.DMA,
            pltpu.SemaphoreType.DMA,
        ],
    )
    def edge_kernel(pack_hbm, tabu_hbm, tabi_hbm, outi_hbm, outu_hbm,
                    pvb, gib, sib, rows, acc, sem_i, sem_g0, sem_g1):
        c = lax.axis_index("c")
        s = lax.axis_index("s")
        r0 = rows.at[0]
        ck0 = s * n_chunks
        row0 = c * HROWS

        def fire_idx(k, b):
            pltpu.async_copy(pack_hbm.at[pl.ds((ck0 + k) * PW, PW)],
                             pvb.at[b], sem_i)

        def wait_idx(b):
            pltpu.make_async_copy(pack_hbm.at[pl.ds(0, PW)], pvb.at[b],
                                  sem_i).wait()

        def run_pass(tab_hbm, out_hbm, goff, soff):
            def zfill(i, _):
                for j in range(D // 16):
                    r0[i, pl.ds(j * 16, 16)] = jnp.zeros((16,), jnp.float32)
                return 0
            lax.fori_loop(0, ZR, zfill, 0)
            for k in range(TSL // ZR):
                pltpu.sync_copy(r0, acc.at[pl.ds(s * TSL + k * ZR, ZR)])
            plsc.subcore_barrier()

            def compute_idx(b):
                for j in range(CH // 16):
                    sl = pl.ds(j * 16, 16)
                    t = pvb[b, pl.ds(2 * SEC + j * 16, 16)]
                    gn = pvb[b, pl.ds(goff + j * 16, 16)]
                    sn = pvb[b, pl.ds(soff + j * 16, 16)]
                    gib[b, sl] = gn * R + t
                    loc = sn * R + t - row0
                    oob = (loc < 0) | (loc >= HROWS)
                    sib[b, sl] = jnp.where(
                        oob, HROWS + (sn & (GROWS - 1)), loc)

            def fire_gather(b, sem):
                pltpu.async_copy(tab_hbm.at[gib.at[b]], rows.at[b], sem)

            def wait_gather(b, sem):
                pltpu.make_async_copy(tab_hbm.at[gib.at[b]], rows.at[b],
                                      sem).wait()

            def scatter(b):
                pltpu.sync_copy(rows.at[b], acc.at[sib.at[b]], add=True)

            M = n_chunks // 2
            fire_idx(0, 0)

            def pair(m, _):
                a = 2 * m
                wait_idx(0)
                compute_idx(0)
                fire_gather(0, sem_g0)
                fire_idx(a + 1, 1)

                @pl.when(m > 0)
                def _():
                    wait_gather(1, sem_g1)
                    scatter(1)
                wait_idx(1)
                compute_idx(1)
                fire_gather(1, sem_g1)

                @pl.when(m < M - 1)
                def _():
                    fire_idx(a + 2, 0)
                wait_gather(0, sem_g0)
                scatter(0)
                return 0
            lax.fori_loop(0, M, pair, 0)
            wait_gather(1, sem_g1)
            scatter(1)
            plsc.subcore_barrier()
            for k in range(TSL // ZR):
                off = s * TSL + k * ZR
                pltpu.sync_copy(acc.at[pl.ds(off, ZR)], r0)
                pltpu.sync_copy(r0, out_hbm.at[c, pl.ds(off, ZR)])

        run_pass(tabu_hbm, outi_hbm, 0, SEC)
        run_pass(tabi_hbm, outu_hbm, SEC, 0)

    return edge_kernel(pack, table_u, table_i)


def _sc_dec_gather(z, pack, n_edges):
    E = n_edges
    per_w = E // 32
    n_chunks = per_w // CH
    PW = 2 * SEC

    @functools.partial(
        pl.kernel,
        out_type=jax.ShapeDtypeStruct((E, D), jnp.float32),
        mesh=_MESH,
        scratch_types=[
            pltpu.VMEM((2, PW), jnp.int32),
            pltpu.VMEM((2, CH), jnp.int32),
            pltpu.VMEM((2, CH), jnp.int32),
            pltpu.VMEM((2, CH, D), jnp.float32),
            pltpu.VMEM((2, CH, D), jnp.float32),
            pltpu.SemaphoreType.DMA,
            pltpu.SemaphoreType.DMA,
            pltpu.SemaphoreType.DMA,
            pltpu.SemaphoreType.DMA,
            pltpu.SemaphoreType.DMA,
        ],
    )
    def dec_kernel(z_hbm, pack_hbm, w_hbm,
                   pvb, svb, dvb, ru, rv, sem_i, sem_u0, sem_u1, sem_v0,
                   sem_v1):
        c = lax.axis_index("c")
        s = lax.axis_index("s")
        w = s * 2 + c
        ck0 = w * n_chunks
        base0 = w * per_w

        def fire_idx(k, b):
            pltpu.async_copy(pack_hbm.at[pl.ds((ck0 + k) * PW, PW)],
                             pvb.at[b], sem_i)

        def wait_idx(b):
            pltpu.make_async_copy(pack_hbm.at[pl.ds(0, PW)], pvb.at[b],
                                  sem_i).wait()
            for j in range(CH // 16):
                sl = pl.ds(j * 16, 16)
                svb[b, sl] = pvb[b, pl.ds(j * 16, 16)]
                dvb[b, sl] = pvb[b, pl.ds(SEC + j * 16, 16)]

        def fire_gathers(b, su, sv_):
            pltpu.async_copy(z_hbm.at[svb.at[b]], ru.at[b], su)
            pltpu.async_copy(z_hbm.at[dvb.at[b]], rv.at[b], sv_)

        def drain_write(k, b, su, sv_):
            base = base0 + k * CH
            pltpu.make_async_copy(z_hbm.at[svb.at[b]], ru.at[b], su).wait()
            pltpu.make_async_copy(z_hbm.at[dvb.at[b]], rv.at[b], sv_).wait()

            def merge(e, _):
                for j in range(HALF // 16):
                    sl = pl.ds(HALF + j * 16, 16)
                    ru[b, e, sl] = rv[b, e, sl]
                return 0
            lax.fori_loop(0, CH, merge, 0)
            pltpu.sync_copy(ru.at[b], w_hbm.at[pl.ds(base, CH)])

        M = n_chunks // 2
        fire_idx(0, 0)

        def pair(m, _):
            a = 2 * m
            wait_idx(0)
            fire_gathers(0, sem_u0, sem_v0)
            fire_idx(a + 1, 1)

            @pl.when(m > 0)
            def _():
                drain_write(a - 1, 1, sem_u1, sem_v1)
            wait_idx(1)
            fire_gathers(1, sem_u1, sem_v1)

            @pl.when(m < M - 1)
            def _():
                fire_idx(a + 2, 0)
            drain_write(a, 0, sem_u0, sem_v0)
            return 0
        lax.fori_loop(0, M, pair, 0)
        drain_write(2 * M - 1, 1, sem_u1, sem_v1)
        if n_chunks % 2:
            k = n_chunks - 1
            fire_idx(k, 0)
            wait_idx(0)
            fire_gathers(0, sem_u0, sem_v0)
            drain_write(k, 0, sem_u0, sem_v0)

    return dec_kernel(z, pack)


def _tc_prep(deg_u, deg_i, ufeats, ifeats):
    NB = 200
    grid = (N // NB,)

    def body(du_ref, di_ref, uf_ref, if_ref,
             su_ref, si_ref, cu_ref, ci_ref):
        cu = lax.rsqrt(jnp.maximum(du_ref[...], 1.0))
        ci = lax.rsqrt(jnp.maximum(di_ref[...], 1.0))
        cu_ref[...] = cu
        ci_ref[...] = ci
        su_ref[...] = cu[:, :, None] * uf_ref[...][:, None, :]
        si_ref[...] = ci[:, :, None] * if_ref[...][:, None, :]

    deg_spec = pl.BlockSpec((NB, R), lambda i: (i, 0))
    feat_spec = pl.BlockSpec((NB, D), lambda i: (i, 0))
    tab_spec = pl.BlockSpec((NB, R, D), lambda i: (i, 0, 0))
    tab_ty = jax.ShapeDtypeStruct((N, R, D), jnp.float32)
    return pl.pallas_call(
        body,
        grid=grid,
        in_specs=[deg_spec, deg_spec, feat_spec, feat_spec],
        out_specs=[tab_spec, tab_spec, deg_spec, deg_spec],
        out_shape=[tab_ty, tab_ty,
                   jax.ShapeDtypeStruct((N, R), jnp.float32),
                   jax.ShapeDtypeStruct((N, R), jnp.float32)],
    )(deg_u, deg_i, ufeats, ifeats)


def _tc_encoder(acc_u, acc_i, cu, ci, W_u, W_i, W_out_u, W_out_i):
    NB = 200
    grid = (N // NB,)

    def body(au_ref, ai_ref, cu_ref, ci_ref, wu_ref, wi_ref,
             wou_ref, woi_ref, zu_ref, zi_ref):
        xu = cu_ref[...][:, :, None] * au_ref[...]
        xi = ci_ref[...][:, :, None] * ai_ref[...]
        wu = wu_ref[...]
        wi = wi_ref[...]
        agg_u = jnp.zeros((NB, HID), jnp.float32)
        agg_i = jnp.zeros((NB, HID), jnp.float32)
        for r in range(R):
            agg_u = agg_u + jax.lax.dot(xu[:, r, :], wi[r],
                                        preferred_element_type=jnp.float32)
            agg_i = agg_i + jax.lax.dot(xi[:, r, :], wu[r],
                                        preferred_element_type=jnp.float32)
        hu = jnp.maximum(agg_u, 0.0)
        hi = jnp.maximum(agg_i, 0.0)
        zu_ref[...] = jnp.maximum(
            jax.lax.dot(hu, wou_ref[...], preferred_element_type=jnp.float32), 0.0)
        zi_ref[...] = jnp.maximum(
            jax.lax.dot(hi, woi_ref[...], preferred_element_type=jnp.float32), 0.0)

    acc_spec = pl.BlockSpec((NB, R, D), lambda i: (i, 0, 0))
    c_spec = pl.BlockSpec((NB, R), lambda i: (i, 0))
    w_spec = pl.BlockSpec((R, D, HID), lambda i: (0, 0, 0))
    wo_spec = pl.BlockSpec((HID, OUT), lambda i: (0, 0))
    z_spec = pl.BlockSpec((NB, OUT), lambda i: (i, 0))
    z_ty = jax.ShapeDtypeStruct((N, OUT), jnp.float32)
    return pl.pallas_call(
        body,
        grid=grid,
        in_specs=[acc_spec, acc_spec, c_spec, c_spec, w_spec, w_spec,
                  wo_spec, wo_spec],
        out_specs=[z_spec, z_spec],
        out_shape=[z_ty, z_ty],
    )(acc_u, acc_i, cu, ci, W_u, W_i, W_out_u, W_out_i)


def _tc_decoder(wuv, Q, coef):
    E = wuv.shape[0]
    EB = 1000
    grid = (E // EB,)

    def body(w_ref, q_ref, coef_ref, out_ref):
        u = w_ref[...][:, :OUT]
        v = w_ref[...][:, OUT:]
        ys = []
        for b in range(2):
            t = jax.lax.dot_general(v, q_ref[b], (((1,), (1,)), ((), ())),
                                    preferred_element_type=jnp.float32)
            ys.append(jnp.sum(u * t, axis=1))
        out_ref[...] = (ys[0][:, None] * coef_ref[:, 0][None, :]
                        + ys[1][:, None] * coef_ref[:, 1][None, :])

    w_spec = pl.BlockSpec((EB, D), lambda i: (i, 0))
    return pl.pallas_call(
        body,
        grid=grid,
        in_specs=[w_spec,
                  pl.BlockSpec((2, OUT, OUT), lambda i: (0, 0, 0)),
                  pl.BlockSpec((R, 2), lambda i: (0, 0))],
        out_specs=pl.BlockSpec((EB, R), lambda i: (i, 0)),
        out_shape=jax.ShapeDtypeStruct((E, R), jnp.float32),
    )(wuv, Q, coef)


def kernel(ufeats, ifeats, enc_edge_index, enc_edge_type, dec_edge_index,
           W_u, W_i, W_out_u, W_out_i, Q, coef):
    src = enc_edge_index[0]
    dst = enc_edge_index[1]
    typ = enc_edge_type
    E = typ.shape[0]
    nck = E // CH

    pad = ((0, 0), (0, SEC - CH))
    enc_pack = jnp.concatenate(
        [jnp.pad(src.reshape(nck, CH), pad),
         jnp.pad(dst.reshape(nck, CH), pad),
         jnp.pad(typ.reshape(nck, CH), pad)], axis=1).reshape(-1)
    ED = dec_edge_index.shape[1]
    dnck = ED // CH
    dec_pack = jnp.concatenate(
        [jnp.pad(dec_edge_index[0].reshape(dnck, CH), pad),
         jnp.pad(dec_edge_index[1].reshape(dnck, CH), pad)],
        axis=1).reshape(-1)

    deg2 = _sc_degrees(enc_pack, E)
    deg_u = deg2[:RN].reshape(N, R)
    deg_i = deg2[ACC:ACC + RN].reshape(N, R)

    table_u, table_i, cu, ci = _tc_prep(deg_u, deg_i, ufeats, ifeats)
    table_u = table_u.reshape(RN, D)
    table_i = table_i.reshape(RN, D)

    acc_i2, acc_u2 = _sc_edge_dual(enc_pack, table_u, table_i, E)
    acc_i = acc_i2.reshape(ACC, D)[:RN].reshape(N, R, D)
    acc_u = acc_u2.reshape(ACC, D)[:RN].reshape(N, R, D)

    zu, zi = _tc_encoder(acc_u, acc_i, cu, ci, W_u, W_i, W_out_u, W_out_i)
    z = jnp.concatenate([zu, zi], axis=1)
    wuv = _sc_dec_gather(z, dec_pack, ED)
    return _tc_decoder(wuv, Q, coef)


# speedup vs baseline: 28.7444x; 1.0311x over previous
import functools

import jax
import jax.numpy as jnp
from jax import lax
from jax.experimental import pallas as pl
from jax.experimental.pallas import tpu as pltpu
from jax.experimental.pallas import tpu_sc as plsc

R = 5
N = 5000
D = 128
HALF = 64
HID = 256
OUT = 64
RN = R * N
ACC = 25600
HROWS = ACC // 2
GROWS = 256
TSL = HROWS // 16
CH = 80
SEC = 128
ZR = 80

_MESH = plsc.VectorSubcoreMesh(core_axis_name="c", subcore_axis_name="s")


def _sc_degrees(pack, n_edges):
    E = n_edges
    per_tile = E // 16
    n_chunks = per_tile // CH
    DSL = ACC // 16
    PW = 3 * SEC

    @functools.partial(
        pl.kernel,
        out_type=jax.ShapeDtypeStruct((2 * ACC,), jnp.float32),
        mesh=_MESH,
        scratch_types=[
            pltpu.VMEM((2, PW), jnp.int32),
            pltpu.VMEM((2, CH), jnp.int32),
            pltpu.VMEM((CH,), jnp.float32),
            pltpu.VMEM((DSL,), jnp.float32),
            pltpu.VMEM_SHARED((ACC,), jnp.float32),
            pltpu.SemaphoreType.DMA,
        ],
    )
    def deg_kernel(pack_hbm, out_hbm, pvb, ivb, ones, zrow, acc, sem_i):
        c = lax.axis_index("c")
        s = lax.axis_index("s")

        def zfill(i, _):
            zrow[pl.ds(i * 16, 16)] = jnp.zeros((16,), jnp.float32)
            return 0
        lax.fori_loop(0, DSL // 16, zfill, 0)
        for j in range(CH // 16):
            ones[pl.ds(j * 16, 16)] = jnp.ones((16,), jnp.float32)
        pltpu.sync_copy(zrow, acc.at[pl.ds(s * DSL, DSL)])
        plsc.subcore_barrier()

        ck0 = s * n_chunks
        noff = c * SEC

        def fire_idx(k, b):
            pltpu.async_copy(pack_hbm.at[pl.ds((ck0 + k) * PW, PW)],
                             pvb.at[b], sem_i)

        def wait_idx(b):
            pltpu.make_async_copy(pack_hbm.at[pl.ds(0, PW)], pvb.at[b],
                                  sem_i).wait()

        def do_chunk(b):
            for j in range(CH // 16):
                sl = pl.ds(j * 16, 16)
                ivb[b, sl] = (pvb[b, pl.ds(noff + j * 16, 16)] * R
                              + pvb[b, pl.ds(2 * SEC + j * 16, 16)])
            pltpu.sync_copy(ones, acc.at[ivb.at[b]], add=True)

        M = n_chunks // 2
        fire_idx(0, 0)

        def pair(m, _):
            a = 2 * m
            wait_idx(0)
            fire_idx(a + 1, 1)
            do_chunk(0)
            wait_idx(1)

            @pl.when(m < M - 1)
            def _():
                fire_idx(a + 2, 0)
            do_chunk(1)
            return 0
        lax.fori_loop(0, M, pair, 0)
        plsc.subcore_barrier()
        pltpu.sync_copy(acc.at[pl.ds(s * DSL, DSL)], zrow)
        pltpu.sync_copy(zrow, out_hbm.at[pl.ds(c * ACC + s * DSL, DSL)])

    return deg_kernel(pack)


def _sc_edge_dual(pack, table_u, table_i, n_edges):
    E = n_edges
    per_tile = E // 16
    n_chunks = per_tile // CH
    PW = 3 * SEC

    @functools.partial(
        pl.kernel,
        out_type=[jax.ShapeDtypeStruct((2, HROWS, D), jnp.float32),
                  jax.ShapeDtypeStruct((2, HROWS, D), jnp.float32)],
        mesh=_MESH,
        scratch_types=[
            pltpu.VMEM((2, PW), jnp.int32),
            pltpu.VMEM((2, CH), jnp.int32),
            pltpu.VMEM((2, CH), jnp.int32),
            pltpu.VMEM((2, CH, D), jnp.float32),
            pltpu.VMEM_SHARED((HROWS + GROWS, D), jnp.float32),
            pltpu.SemaphoreType.DMA,
            pltpu.SemaphoreType.DMA,
            pltpu.SemaphoreType.DMA,
        ],
    )
    def edge_kernel(pack_hbm, tabu_hbm, tabi_hbm, outi_hbm, outu_hbm,
                    pvb, gib, sib, rows, acc, sem_i, sem_g0, sem_g1):
        c = lax.axis_index("c")
        s = lax.axis_index("s")
        r0 = rows.at[0]
        ck0 = s * n_chunks
        row0 = c * HROWS

        def fire_idx(k, b):
            pltpu.async_copy(pack_hbm.at[pl.ds((ck0 + k) * PW, PW)],
                             pvb.at[b], sem_i)

        def wait_idx(b):
            pltpu.make_async_copy(pack_hbm.at[pl.ds(0, PW)], pvb.at[b],
                                  sem_i).wait()

        def run_pass(tab_hbm, out_hbm, goff, soff):
            def zfill(i, _):
                for j in range(D // 16):
                    r0[i, pl.ds(j * 16, 16)] = jnp.zeros((16,), jnp.float32)
                return 0
            lax.fori_loop(0, ZR, zfill, 0)
            for k in range(TSL // ZR):
                pltpu.sync_copy(r0, acc.at[pl.ds(s * TSL + k * ZR, ZR)])
            plsc.subcore_barrier()

            def compute_idx(b):
                for j in range(CH // 16):
                    sl = pl.ds(j * 16, 16)
                    t = pvb[b, pl.ds(2 * SEC + j * 16, 16)]
                    gn = pvb[b, pl.ds(goff + j * 16, 16)]
                    sn = pvb[b, pl.ds(soff + j * 16, 16)]
                    gib[b, sl] = gn * R + t
                    loc = sn * R + t - row0
                    oob = (loc < 0) | (loc >= HROWS)
                    sib[b, sl] = jnp.where(
                        oob, HROWS + (sn & (GROWS - 1)), loc)

            def fire_gather(b, sem):
                pltpu.async_copy(tab_hbm.at[gib.at[b]], rows.at[b], sem)

            def wait_gather(b, sem):
                pltpu.make_async_copy(tab_hbm.at[gib.at[b]], rows.at[b],
                                      sem).wait()

            def scatter(b):
                pltpu.sync_copy(rows.at[b], acc.at[sib.at[b]], add=True)

            M = n_chunks // 2
            fire_idx(0, 0)

            def pair(m, _):
                a = 2 * m
                wait_idx(0)
                compute_idx(0)
                fire_gather(0, sem_g0)
                fire_idx(a + 1, 1)

                @pl.when(m > 0)
                def _():
                    wait_gather(1, sem_g1)
                    scatter(1)
                wait_idx(1)
                compute_idx(1)
                fire_gather(1, sem_g1)

                @pl.when(m < M - 1)
                def _():
                    fire_idx(a + 2, 0)
                wait_gather(0, sem_g0)
                scatter(0)
                return 0
            lax.fori_loop(0, M, pair, 0)
            wait_gather(1, sem_g1)
            scatter(1)
            plsc.subcore_barrier()
            for k in range(TSL // ZR):
                off = s * TSL + k * ZR
                pltpu.sync_copy(acc.at[pl.ds(off, ZR)], r0)
                pltpu.sync_copy(r0, out_hbm.at[c, pl.ds(off, ZR)])

        run_pass(tabu_hbm, outi_hbm, 0, SEC)
        run_pass(tabi_hbm, outu_hbm, SEC, 0)

    return edge_kernel(pack, table_u, table_i)


def _sc_dec_gather(z, pack, n_edges):
    E = n_edges
    per_w = E // 32
    n_chunks = per_w // CH
    PW = 2 * SEC

    @functools.partial(
        pl.kernel,
        out_type=jax.ShapeDtypeStruct((E, D), jnp.float32),
        mesh=_MESH,
        scratch_types=[
            pltpu.VMEM((2, PW), jnp.int32),
            pltpu.VMEM((2, CH), jnp.int32),
            pltpu.VMEM((2, CH), jnp.int32),
            pltpu.VMEM((2, CH, D), jnp.float32),
            pltpu.VMEM((2, CH, D), jnp.float32),
            pltpu.SemaphoreType.DMA,
            pltpu.SemaphoreType.DMA,
            pltpu.SemaphoreType.DMA,
            pltpu.SemaphoreType.DMA,
            pltpu.SemaphoreType.DMA,
        ],
    )
    def dec_kernel(z_hbm, pack_hbm, w_hbm,
                   pvb, svb, dvb, ru, rv, sem_i, sem_u0, sem_u1, sem_v0,
                   sem_v1):
        c = lax.axis_index("c")
        s = lax.axis_index("s")
        w = s * 2 + c
        ck0 = w * n_chunks
        base0 = w * per_w

        def fire_idx(k, b):
            pltpu.async_copy(pack_hbm.at[pl.ds((ck0 + k) * PW, PW)],
                             pvb.at[b], sem_i)

        def wait_idx(b):
            pltpu.make_async_copy(pack_hbm.at[pl.ds(0, PW)], pvb.at[b],
                                  sem_i).wait()
            for j in range(CH // 16):
                sl = pl.ds(j * 16, 16)
                svb[b, sl] = pvb[b, pl.ds(j * 16, 16)]
                dvb[b, sl] = pvb[b, pl.ds(SEC + j * 16, 16)]

        def fire_gathers(b, su, sv_):
            pltpu.async_copy(z_hbm.at[svb.at[b]], ru.at[b], su)
            pltpu.async_copy(z_hbm.at[dvb.at[b]], rv.at[b], sv_)

        def drain_write(k, b, su, sv_):
            base = base0 + k * CH
            pltpu.make_async_copy(z_hbm.at[svb.at[b]], ru.at[b], su).wait()
            pltpu.make_async_copy(z_hbm.at[dvb.at[b]], rv.at[b], sv_).wait()

            def merge(e, _):
                for j in range(HALF // 16):
                    sl = pl.ds(HALF + j * 16, 16)
                    ru[b, e, sl] = rv[b, e, sl]
                return 0
            lax.fori_loop(0, CH, merge, 0)
            pltpu.sync_copy(ru.at[b], w_hbm.at[pl.ds(base, CH)])

        M = n_chunks // 2
        fire_idx(0, 0)

        def pair(m, _):
            a = 2 * m
            wait_idx(0)
            fire_gathers(0, sem_u0, sem_v0)
            fire_idx(a + 1, 1)

            @pl.when(m > 0)
            def _():
                drain_write(a - 1, 1, sem_u1, sem_v1)
            wait_idx(1)
            fire_gathers(1, sem_u1, sem_v1)

            @pl.when(m < M - 1)
            def _():
                fire_idx(a + 2, 0)
            drain_write(a, 0, sem_u0, sem_v0)
            return 0
        lax.fori_loop(0, M, pair, 0)
        drain_write(2 * M - 1, 1, sem_u1, sem_v1)
        if n_chunks % 2:
            k = n_chunks - 1
            fire_idx(k, 0)
            wait_idx(0)
            fire_gathers(0, sem_u0, sem_v0)
            drain_write(k, 0, sem_u0, sem_v0)

    return dec_kernel(z, pack)


def _tc_prep(deg_u, deg_i, ufeats, ifeats):
    NB = 200
    grid = (N // NB,)

    def body(du_ref, di_ref, uf_ref, if_ref,
             su_ref, si_ref, cu_ref, ci_ref):
        cu = lax.rsqrt(jnp.maximum(du_ref[...], 1.0))
        ci = lax.rsqrt(jnp.maximum(di_ref[...], 1.0))
        cu_ref[...] = cu
        ci_ref[...] = ci
        su_ref[...] = cu[:, :, None] * uf_ref[...][:, None, :]
        si_ref[...] = ci[:, :, None] * if_ref[...][:, None, :]

    deg_spec = pl.BlockSpec((NB, R), lambda i: (i, 0))
    feat_spec = pl.BlockSpec((NB, D), lambda i: (i, 0))
    tab_spec = pl.BlockSpec((NB, R, D), lambda i: (i, 0, 0))
    tab_ty = jax.ShapeDtypeStruct((N, R, D), jnp.float32)
    return pl.pallas_call(
        body,
        grid=grid,
        in_specs=[deg_spec, deg_spec, feat_spec, feat_spec],
        out_specs=[tab_spec, tab_spec, deg_spec, deg_spec],
        out_shape=[tab_ty, tab_ty,
                   jax.ShapeDtypeStruct((N, R), jnp.float32),
                   jax.ShapeDtypeStruct((N, R), jnp.float32)],
    )(deg_u, deg_i, ufeats, ifeats)


def _tc_encoder(acc_u, acc_i, cu, ci, W_u, W_i, W_out_u, W_out_i):
    NB = 200
    grid = (N // NB,)

    def body(au_ref, ai_ref, cu_ref, ci_ref, wu_ref, wi_ref,
             wou_ref, woi_ref, z_ref):
        au = au_ref[...].reshape(NB, R, D)
        ai = ai_ref[...].reshape(NB, R, D)
        xu = cu_ref[...][:, :, None] * au
        xi = ci_ref[...][:, :, None] * ai
        wu = wu_ref[...]
        wi = wi_ref[...]
        agg_u = jnp.zeros((NB, HID), jnp.float32)
        agg_i = jnp.zeros((NB, HID), jnp.float32)
        for r in range(R):
            agg_u = agg_u + jax.lax.dot(xu[:, r, :], wi[r],
                                        preferred_element_type=jnp.float32)
            agg_i = agg_i + jax.lax.dot(xi[:, r, :], wu[r],
                                        preferred_element_type=jnp.float32)
        hu = jnp.maximum(agg_u, 0.0)
        hi = jnp.maximum(agg_i, 0.0)
        zu = jnp.maximum(
            jax.lax.dot(hu, wou_ref[...], preferred_element_type=jnp.float32), 0.0)
        zi = jnp.maximum(
            jax.lax.dot(hi, woi_ref[...], preferred_element_type=jnp.float32), 0.0)
        z_ref[...] = jnp.concatenate([zu, zi], axis=1)

    acc_spec = pl.BlockSpec((NB * R, D), lambda i: (i, 0))
    c_spec = pl.BlockSpec((NB, R), lambda i: (i, 0))
    w_spec = pl.BlockSpec((R, D, HID), lambda i: (0, 0, 0))
    wo_spec = pl.BlockSpec((HID, OUT), lambda i: (0, 0))
    z_spec = pl.BlockSpec((NB, D), lambda i: (i, 0))
    z_ty = jax.ShapeDtypeStruct((N, D), jnp.float32)
    return pl.pallas_call(
        body,
        grid=grid,
        in_specs=[acc_spec, acc_spec, c_spec, c_spec, w_spec, w_spec,
                  wo_spec, wo_spec],
        out_specs=z_spec,
        out_shape=z_ty,
    )(acc_u, acc_i, cu, ci, W_u, W_i, W_out_u, W_out_i)


def _tc_decoder(wuv, Q, coef):
    E = wuv.shape[0]
    EB = 1000
    grid = (E // EB,)

    def body(w_ref, q_ref, coef_ref, out_ref):
        u = w_ref[...][:, :OUT]
        v = w_ref[...][:, OUT:]
        ys = []
        for b in range(2):
            t = jax.lax.dot_general(v, q_ref[b], (((1,), (1,)), ((), ())),
                                    preferred_element_type=jnp.float32)
            ys.append(jnp.sum(u * t, axis=1))
        out_ref[...] = (ys[0][:, None] * coef_ref[:, 0][None, :]
                        + ys[1][:, None] * coef_ref[:, 1][None, :])

    w_spec = pl.BlockSpec((EB, D), lambda i: (i, 0))
    return pl.pallas_call(
        body,
        grid=grid,
        in_specs=[w_spec,
                  pl.BlockSpec((2, OUT, OUT), lambda i: (0, 0, 0)),
                  pl.BlockSpec((R, 2), lambda i: (0, 0))],
        out_specs=pl.BlockSpec((EB, R), lambda i: (i, 0)),
        out_shape=jax.ShapeDtypeStruct((E, R), jnp.float32),
    )(wuv, Q, coef)


def kernel(ufeats, ifeats, enc_edge_index, enc_edge_type, dec_edge_index,
           W_u, W_i, W_out_u, W_out_i, Q, coef):
    src = enc_edge_index[0]
    dst = enc_edge_index[1]
    typ = enc_edge_type
    E = typ.shape[0]
    nck = E // CH

    pad = ((0, 0), (0, SEC - CH))
    enc_pack = jnp.concatenate(
        [jnp.pad(src.reshape(nck, CH), pad),
         jnp.pad(dst.reshape(nck, CH), pad),
         jnp.pad(typ.reshape(nck, CH), pad)], axis=1).reshape(-1)
    ED = dec_edge_index.shape[1]
    dnck = ED // CH
    dec_pack = jnp.concatenate(
        [jnp.pad(dec_edge_index[0].reshape(dnck, CH), pad),
         jnp.pad(dec_edge_index[1].reshape(dnck, CH), pad)],
        axis=1).reshape(-1)

    deg2 = _sc_degrees(enc_pack, E)
    deg_u = deg2[:RN].reshape(N, R)
    deg_i = deg2[ACC:ACC + RN].reshape(N, R)

    table_u, table_i, cu, ci = _tc_prep(deg_u, deg_i, ufeats, ifeats)
    table_u = table_u.reshape(RN, D)
    table_i = table_i.reshape(RN, D)

    acc_i2, acc_u2 = _sc_edge_dual(enc_pack, table_u, table_i, E)
    acc_i = acc_i2.reshape(ACC, D)
    acc_u = acc_u2.reshape(ACC, D)

    z = _tc_encoder(acc_u, acc_i, cu, ci, W_u, W_i, W_out_u, W_out_i)
    wuv = _sc_dec_gather(z, dec_pack, ED)
    return _tc_decoder(wuv, Q, coef)
